# baseline (speedup 1.0000x reference)
import functools

import jax
import jax.numpy as jnp
from jax import lax
from jax.experimental import pallas as pl
from jax.experimental.pallas import tpu as pltpu
from jax.experimental.pallas import tpu_sc as plsc

EMBED_DIM = 64
PAD_DIM = 128
BATCH = 16384
HIST = 50

B = BATCH * HIST
NC, NS = 2, 16
NW = NC * NS
PER_W = B // NW
CH = 128
NCHUNK = PER_W // CH
NBUF = 4
NMACRO = NCHUNK // NBUF


def _make_kernel():
    mesh = plsc.VectorSubcoreMesh(core_axis_name="c", subcore_axis_name="s")

    @functools.partial(
        pl.kernel,
        mesh=mesh,
        out_type=jax.ShapeDtypeStruct((B, PAD_DIM), jnp.float32),
        scratch_types=[
            pltpu.VMEM((NCHUNK, CH), jnp.int32),
            pltpu.VMEM((NBUF, CH, PAD_DIM), jnp.float32),
            [pltpu.SemaphoreType.DMA] * NBUF,
            [pltpu.SemaphoreType.DMA] * NBUF,
        ],
        compiler_params=pltpu.CompilerParams(use_tc_tiling_on_sc=False),
    )
    def k(idx_hbm, table_hbm, out_hbm, idx_v, rows_v, gsems, wsems):
        wid = lax.axis_index("s") * NC + lax.axis_index("c")
        base = wid * PER_W
        pltpu.sync_copy(idx_hbm.at[wid], idx_v)

        def gather_start(g, b):
            pltpu.async_copy(table_hbm.at[idx_v.at[g]], rows_v.at[b], gsems[b])

        def gather_wait(g, b):
            pltpu.make_async_copy(
                table_hbm.at[idx_v.at[g]], rows_v.at[b], gsems[b]
            ).wait()

        def write_start(g, b):
            pltpu.async_copy(
                rows_v.at[b], out_hbm.at[pl.ds(base + g * CH, CH)], wsems[b]
            )

        def write_wait(g, b):
            pltpu.make_async_copy(
                rows_v.at[b], out_hbm.at[pl.ds(base + g * CH, CH)], wsems[b]
            ).wait()

        for b in range(NBUF):
            gather_start(b, b)

        for b in range(NBUF):
            gather_wait(b, b)
            write_start(b, b)
            if b >= 1:
                write_wait(b - 1, b - 1)
                gather_start(b - 1 + NBUF, b - 1)

        def macro(kk, carry):
            g0 = kk * NBUF
            for b in range(NBUF):
                g = g0 + b
                gather_wait(g, b)
                write_start(g, b)
                bp = (b - 1) % NBUF
                write_wait(g - 1, bp)
                gather_start(g - 1 + NBUF, bp)
            return carry

        lax.fori_loop(1, NMACRO - 1, macro, 0)

        g0 = NCHUNK - NBUF
        for b in range(NBUF):
            g = g0 + b
            gather_wait(g, b)
            write_start(g, b)
            if b == 0:
                bp = (b - 1) % NBUF
                write_wait(g - 1, bp)
                gather_start(NCHUNK - 1, bp)

        for b in range(NBUF):
            write_wait(g0 + b, b)

    return k


_gather_kernel = _make_kernel()


def kernel(item_ids, table):
    table_pad = jnp.pad(table, ((0, 0), (0, PAD_DIM - EMBED_DIM)))
    idx = item_ids.reshape(NW, NCHUNK, CH).astype(jnp.int32)
    out_pad = _gather_kernel(idx, table_pad)
    return out_pad[:, :EMBED_DIM].reshape(BATCH, HIST, EMBED_DIM)


# speedup vs baseline: 1.4039x; 1.4039x over previous
import functools

import jax
import jax.numpy as jnp
from jax import lax
from jax.experimental import pallas as pl
from jax.experimental.pallas import tpu as pltpu
from jax.experimental.pallas import tpu_sc as plsc

EMBED_DIM = 64
PAD_DIM = 128
BATCH = 16384
HIST = 50
HIST_PAD = 56

B = BATCH * HIST
NC, NS = 2, 16
NW = NC * NS
PER_W = B // NW
BPW = BATCH // NW
CH = 2 * HIST
NCHUNK = PER_W // CH
NBUF = 4
NMACRO = NCHUNK // NBUF


def _make_kernel():
    mesh = plsc.VectorSubcoreMesh(core_axis_name="c", subcore_axis_name="s")

    @functools.partial(
        pl.kernel,
        mesh=mesh,
        out_type=jax.ShapeDtypeStruct((BATCH, HIST_PAD, PAD_DIM), jnp.float32),
        scratch_types=[
            pltpu.VMEM((NCHUNK, CH), jnp.int32),
            pltpu.VMEM((NBUF, CH, PAD_DIM), jnp.float32),
            [pltpu.SemaphoreType.DMA] * NBUF,
            [pltpu.SemaphoreType.DMA] * NBUF,
        ],
        compiler_params=pltpu.CompilerParams(use_tc_tiling_on_sc=False),
    )
    def k(idx_hbm, table_hbm, out_hbm, idx_v, rows_v, gsems, wsems):
        wid = lax.axis_index("s") * NC + lax.axis_index("c")
        base_b = wid * BPW
        pltpu.sync_copy(idx_hbm.at[wid], idx_v)

        def gather_start(g, b):
            pltpu.async_copy(table_hbm.at[idx_v.at[g]], rows_v.at[b], gsems[b])

        def gather_wait(g, b):
            pltpu.make_async_copy(
                table_hbm.at[idx_v.at[g]], rows_v.at[b], gsems[b]
            ).wait()

        def write_start(g, b):
            b0 = base_b + 2 * g
            pltpu.async_copy(
                rows_v.at[b].at[pl.ds(0, HIST)],
                out_hbm.at[b0, pl.ds(0, HIST)],
                wsems[b],
            )
            pltpu.async_copy(
                rows_v.at[b].at[pl.ds(HIST, HIST)],
                out_hbm.at[b0 + 1, pl.ds(0, HIST)],
                wsems[b],
            )

        def write_wait(g, b):
            b0 = base_b + 2 * g
            pltpu.make_async_copy(
                rows_v.at[b].at[pl.ds(0, HIST)],
                out_hbm.at[b0, pl.ds(0, HIST)],
                wsems[b],
            ).wait()
            pltpu.make_async_copy(
                rows_v.at[b].at[pl.ds(HIST, HIST)],
                out_hbm.at[b0 + 1, pl.ds(0, HIST)],
                wsems[b],
            ).wait()

        for b in range(NBUF):
            gather_start(b, b)

        for b in range(NBUF):
            gather_wait(b, b)
            write_start(b, b)
            if b >= 1:
                write_wait(b - 1, b - 1)
                gather_start(b - 1 + NBUF, b - 1)

        def macro(kk, carry):
            g0 = kk * NBUF
            for b in range(NBUF):
                g = g0 + b
                gather_wait(g, b)
                write_start(g, b)
                bp = (b - 1) % NBUF
                write_wait(g - 1, bp)
                gather_start(g - 1 + NBUF, bp)
            return carry

        lax.fori_loop(1, NMACRO - 1, macro, 0)

        g0 = NCHUNK - NBUF
        for b in range(NBUF):
            g = g0 + b
            gather_wait(g, b)
            write_start(g, b)
            if b == 0:
                bp = (b - 1) % NBUF
                write_wait(g - 1, bp)
                gather_start(NCHUNK - 1, bp)

        for b in range(NBUF):
            write_wait(g0 + b, b)

    return k


_gather_kernel = _make_kernel()


def kernel(item_ids, table):
    table_pad = jnp.pad(table, ((0, 0), (0, PAD_DIM - EMBED_DIM)))
    idx = item_ids.reshape(NW, NCHUNK, CH).astype(jnp.int32)
    out_pad = _gather_kernel(idx, table_pad)
    return out_pad[:, :HIST, :EMBED_DIM]


# speedup vs baseline: 1.5974x; 1.1379x over previous
import functools

import jax
import jax.numpy as jnp
from jax import lax
from jax.experimental import pallas as pl
from jax.experimental.pallas import tpu as pltpu
from jax.experimental.pallas import tpu_sc as plsc

EMBED_DIM = 64
PAD_DIM = 128
BATCH = 16384
HIST = 50
HIST_PAD = 56

NUM_ROWS = 1000000
B = BATCH * HIST
NC, NS = 2, 16
NW = NC * NS
PER_W = B // NW
BPW = BATCH // NW
CH = 2 * HIST
NCHUNK = PER_W // CH
NBUF = 4
NMACRO = NCHUNK // NBUF

TCA_COLS = 2048
TCA_GRID = -(-NUM_ROWS // TCA_COLS)


@functools.partial(
    pl.pallas_call,
    grid=(TCA_GRID,),
    in_specs=[pl.BlockSpec((EMBED_DIM, TCA_COLS), lambda i: (0, i))],
    out_specs=pl.BlockSpec((TCA_COLS // 2, PAD_DIM), lambda i: (i, 0)),
    out_shape=jax.ShapeDtypeStruct((NUM_ROWS // 2, PAD_DIM), jnp.float32),
)
def _detile(tT_ref, out_ref):
    y = tT_ref[...].T
    z = y.reshape(TCA_COLS // 2, 2, EMBED_DIM)
    out_ref[:, 0:EMBED_DIM] = z[:, 0, :]
    out_ref[:, EMBED_DIM:PAD_DIM] = z[:, 1, :]


def _make_gather():
    mesh = plsc.VectorSubcoreMesh(core_axis_name="c", subcore_axis_name="s")

    @functools.partial(
        pl.kernel,
        mesh=mesh,
        out_type=jax.ShapeDtypeStruct((BATCH, HIST_PAD, PAD_DIM), jnp.float32),
        scratch_types=[
            pltpu.VMEM((NCHUNK, CH), jnp.int32),
            pltpu.VMEM((NBUF, CH, EMBED_DIM), jnp.float32),
            [pltpu.SemaphoreType.DMA] * NBUF,
            [pltpu.SemaphoreType.DMA] * NBUF,
        ],
        compiler_params=pltpu.CompilerParams(use_tc_tiling_on_sc=False),
    )
    def k(idx_hbm, table_hbm, out_hbm, idx_v, rows_v, gsems, wsems):
        wid = lax.axis_index("s") * NC + lax.axis_index("c")
        base_b = wid * BPW
        pltpu.sync_copy(idx_hbm.at[wid], idx_v)

        def gather_start(g, b):
            pltpu.async_copy(table_hbm.at[idx_v.at[g]], rows_v.at[b], gsems[b])

        def gather_wait(g, b):
            pltpu.make_async_copy(
                table_hbm.at[idx_v.at[g]], rows_v.at[b], gsems[b]
            ).wait()

        def write_starts(g, b):
            b0 = base_b + 2 * g
            c1 = pltpu.make_async_copy(
                rows_v.at[b].at[pl.ds(0, HIST)],
                out_hbm.at[b0, pl.ds(0, HIST), pl.ds(0, EMBED_DIM)],
                wsems[b],
            )
            c2 = pltpu.make_async_copy(
                rows_v.at[b].at[pl.ds(HIST, HIST)],
                out_hbm.at[b0 + 1, pl.ds(0, HIST), pl.ds(0, EMBED_DIM)],
                wsems[b],
            )
            return c1, c2

        def write_start(g, b):
            c1, c2 = write_starts(g, b)
            c1.start()
            c2.start()

        def write_wait(g, b):
            c1, c2 = write_starts(g, b)
            c1.wait()
            c2.wait()

        for b in range(NBUF):
            gather_start(b, b)

        for b in range(NBUF):
            gather_wait(b, b)
            write_start(b, b)
            if b >= 1:
                write_wait(b - 1, b - 1)
                gather_start(b - 1 + NBUF, b - 1)

        def macro(kk, carry):
            g0 = kk * NBUF
            for b in range(NBUF):
                g = g0 + b
                gather_wait(g, b)
                write_start(g, b)
                bp = (b - 1) % NBUF
                write_wait(g - 1, bp)
                gather_start(g - 1 + NBUF, bp)
            return carry

        lax.fori_loop(1, NMACRO - 1, macro, 0)

        g0 = NCHUNK - NBUF
        for b in range(NBUF):
            g = g0 + b
            gather_wait(g, b)
            write_start(g, b)
            if b == 0:
                bp = (b - 1) % NBUF
                write_wait(g - 1, bp)
                gather_start(NCHUNK - 1, bp)

        for b in range(NBUF):
            write_wait(g0 + b, b)

    return k


_gather_kernel = _make_gather()


def kernel(item_ids, table):
    tlin = _detile(table.T)
    table_lin = tlin.reshape(NUM_ROWS, EMBED_DIM)
    idx = item_ids.reshape(NW, NCHUNK, CH).astype(jnp.int32)
    out_pad = _gather_kernel(idx, table_lin)
    return out_pad[:, :HIST, :EMBED_DIM]


# speedup vs baseline: 1.8509x; 1.1587x over previous
import functools

import jax
import jax.numpy as jnp
from jax import lax
from jax.experimental import pallas as pl
from jax.experimental.pallas import tpu as pltpu
from jax.experimental.pallas import tpu_sc as plsc

EMBED_DIM = 64
PAD_DIM = 128
BATCH = 16384
HIST = 50
HIST_PAD = 56

NUM_ROWS = 1000000
B = BATCH * HIST
NC, NS = 2, 16
NW = NC * NS
PER_W = B // NW
BPW = BATCH // NW
CH = 2 * HIST
NCHUNK = PER_W // CH
NBUF = 4
NMACRO = NCHUNK // NBUF

TCA_COLS = 4096
TCA_GRID = -(-NUM_ROWS // TCA_COLS)


@functools.partial(
    pl.pallas_call,
    grid=(TCA_GRID,),
    in_specs=[pl.BlockSpec((EMBED_DIM, TCA_COLS), lambda i: (0, i))],
    out_specs=pl.BlockSpec((TCA_COLS // 2, PAD_DIM), lambda i: (i, 0)),
    out_shape=jax.ShapeDtypeStruct((NUM_ROWS // 2, PAD_DIM), jnp.float32),
)
def _detile(tT_ref, out_ref):
    y = tT_ref[...].T
    z = y.reshape(TCA_COLS // 2, 2, EMBED_DIM)
    out_ref[...] = jnp.concatenate([z[:, 0, :], z[:, 1, :]], axis=1)


def _make_gather():
    mesh = plsc.VectorSubcoreMesh(core_axis_name="c", subcore_axis_name="s")

    @functools.partial(
        pl.kernel,
        mesh=mesh,
        out_type=jax.ShapeDtypeStruct((BATCH, HIST_PAD, PAD_DIM), jnp.float32),
        scratch_types=[
            pltpu.VMEM((NCHUNK, CH), jnp.int32),
            pltpu.VMEM((NBUF, CH, EMBED_DIM), jnp.float32),
            [pltpu.SemaphoreType.DMA] * NBUF,
            [pltpu.SemaphoreType.DMA] * NBUF,
        ],
        compiler_params=pltpu.CompilerParams(use_tc_tiling_on_sc=False),
    )
    def k(idx_hbm, table_hbm, out_hbm, idx_v, rows_v, gsems, wsems):
        wid = lax.axis_index("s") * NC + lax.axis_index("c")
        base_b = wid * BPW
        pltpu.sync_copy(idx_hbm.at[wid], idx_v)

        def gather_start(g, b):
            pltpu.async_copy(table_hbm.at[idx_v.at[g]], rows_v.at[b], gsems[b])

        def gather_wait(g, b):
            pltpu.make_async_copy(
                table_hbm.at[idx_v.at[g]], rows_v.at[b], gsems[b]
            ).wait()

        def write_starts(g, b):
            b0 = base_b + 2 * g
            c1 = pltpu.make_async_copy(
                rows_v.at[b].at[pl.ds(0, HIST)],
                out_hbm.at[b0, pl.ds(0, HIST), pl.ds(0, EMBED_DIM)],
                wsems[b],
            )
            c2 = pltpu.make_async_copy(
                rows_v.at[b].at[pl.ds(HIST, HIST)],
                out_hbm.at[b0 + 1, pl.ds(0, HIST), pl.ds(0, EMBED_DIM)],
                wsems[b],
            )
            return c1, c2

        def write_start(g, b):
            c1, c2 = write_starts(g, b)
            c1.start()
            c2.start()

        def write_wait(g, b):
            c1, c2 = write_starts(g, b)
            c1.wait()
            c2.wait()

        for b in range(NBUF):
            gather_start(b, b)

        for b in range(NBUF):
            gather_wait(b, b)
            write_start(b, b)
            if b >= 1:
                write_wait(b - 1, b - 1)
                gather_start(b - 1 + NBUF, b - 1)

        def macro(kk, carry):
            g0 = kk * NBUF
            for b in range(NBUF):
                g = g0 + b
                gather_wait(g, b)
                write_start(g, b)
                bp = (b - 1) % NBUF
                write_wait(g - 1, bp)
                gather_start(g - 1 + NBUF, bp)
            return carry

        lax.fori_loop(1, NMACRO - 1, macro, 0)

        g0 = NCHUNK - NBUF
        for b in range(NBUF):
            g = g0 + b
            gather_wait(g, b)
            write_start(g, b)
            if b == 0:
                bp = (b - 1) % NBUF
                write_wait(g - 1, bp)
                gather_start(NCHUNK - 1, bp)

        for b in range(NBUF):
            write_wait(g0 + b, b)

    return k


_gather_kernel = _make_gather()


def kernel(item_ids, table):
    tlin = _detile(table.T)
    table_lin = tlin.reshape(NUM_ROWS, EMBED_DIM)
    idx = item_ids.reshape(NW, NCHUNK, CH).astype(jnp.int32)
    out_pad = _gather_kernel(idx, table_lin)
    return out_pad[:, :HIST, :EMBED_DIM]


# speedup vs baseline: 1.9007x; 1.0269x over previous
import functools

import jax
import jax.numpy as jnp
from jax import lax
from jax.experimental import pallas as pl
from jax.experimental.pallas import tpu as pltpu
from jax.experimental.pallas import tpu_sc as plsc

EMBED_DIM = 64
PAD_DIM = 128
BATCH = 16384
HIST = 50
HIST_PAD = 56

NUM_ROWS = 1000000
B = BATCH * HIST
NC, NS = 2, 16
NW = NC * NS
PER_W = B // NW
BPW = BATCH // NW
CH = 2 * HIST
NCHUNK = PER_W // CH
NBUF = 4
NMACRO = NCHUNK // NBUF

TCA_COLS = 8192
TCA_GRID = -(-NUM_ROWS // TCA_COLS)


@functools.partial(
    pl.pallas_call,
    grid=(TCA_GRID,),
    in_specs=[pl.BlockSpec((EMBED_DIM, TCA_COLS), lambda i: (0, i))],
    out_specs=pl.BlockSpec((TCA_COLS // 2, PAD_DIM), lambda i: (i, 0)),
    out_shape=jax.ShapeDtypeStruct((NUM_ROWS // 2, PAD_DIM), jnp.float32),
)
def _detile(tT_ref, out_ref):
    y = tT_ref[...].T
    z = y.reshape(TCA_COLS // 2, 2, EMBED_DIM)
    out_ref[...] = jnp.concatenate([z[:, 0, :], z[:, 1, :]], axis=1)


def _make_gather():
    mesh = plsc.VectorSubcoreMesh(core_axis_name="c", subcore_axis_name="s")

    @functools.partial(
        pl.kernel,
        mesh=mesh,
        out_type=jax.ShapeDtypeStruct((BATCH, HIST_PAD, PAD_DIM), jnp.float32),
        scratch_types=[
            pltpu.VMEM((NCHUNK, CH), jnp.int32),
            pltpu.VMEM((NBUF, CH, EMBED_DIM), jnp.float32),
            [pltpu.SemaphoreType.DMA] * NBUF,
            [pltpu.SemaphoreType.DMA] * NBUF,
        ],
        compiler_params=pltpu.CompilerParams(use_tc_tiling_on_sc=False),
    )
    def k(idx_hbm, table_hbm, out_hbm, idx_v, rows_v, gsems, wsems):
        wid = lax.axis_index("s") * NC + lax.axis_index("c")
        base_b = wid * BPW
        pltpu.sync_copy(idx_hbm.at[wid], idx_v)

        def gather_start(g, b):
            pltpu.async_copy(table_hbm.at[idx_v.at[g]], rows_v.at[b], gsems[b])

        def gather_wait(g, b):
            pltpu.make_async_copy(
                table_hbm.at[idx_v.at[g]], rows_v.at[b], gsems[b]
            ).wait()

        def write_starts(g, b):
            b0 = base_b + 2 * g
            c1 = pltpu.make_async_copy(
                rows_v.at[b].at[pl.ds(0, HIST)],
                out_hbm.at[b0, pl.ds(0, HIST), pl.ds(0, EMBED_DIM)],
                wsems[b],
            )
            c2 = pltpu.make_async_copy(
                rows_v.at[b].at[pl.ds(HIST, HIST)],
                out_hbm.at[b0 + 1, pl.ds(0, HIST), pl.ds(0, EMBED_DIM)],
                wsems[b],
            )
            return c1, c2

        def write_start(g, b):
            c1, c2 = write_starts(g, b)
            c1.start()
            c2.start()

        def write_wait(g, b):
            c1, c2 = write_starts(g, b)
            c1.wait()
            c2.wait()

        for b in range(NBUF):
            gather_start(b, b)

        for b in range(NBUF):
            gather_wait(b, b)
            write_start(b, b)
            if b >= 1:
                write_wait(b - 1, b - 1)
                gather_start(b - 1 + NBUF, b - 1)

        def macro(kk, carry):
            g0 = kk * NBUF
            for b in range(NBUF):
                g = g0 + b
                gather_wait(g, b)
                write_start(g, b)
                bp = (b - 1) % NBUF
                write_wait(g - 1, bp)
                gather_start(g - 1 + NBUF, bp)
            return carry

        lax.fori_loop(1, NMACRO - 1, macro, 0)

        g0 = NCHUNK - NBUF
        for b in range(NBUF):
            g = g0 + b
            gather_wait(g, b)
            write_start(g, b)
            if b == 0:
                bp = (b - 1) % NBUF
                write_wait(g - 1, bp)
                gather_start(NCHUNK - 1, bp)

        for b in range(NBUF):
            write_wait(g0 + b, b)

    return k


_gather_kernel = _make_gather()


def kernel(item_ids, table):
    tlin = _detile(table.T)
    table_lin = tlin.reshape(NUM_ROWS, EMBED_DIM)
    idx = item_ids.reshape(NW, NCHUNK, CH).astype(jnp.int32)
    out_pad = _gather_kernel(idx, table_lin)
    return out_pad[:, :HIST, :EMBED_DIM]
